# Initial kernel scaffold; baseline (speedup 1.0000x reference)
import jax
import jax.numpy as jnp
from jax.experimental import pallas as pl


def kernel(x, enc_w1, enc_b1, enc_w2, enc_b2, codebook, dec_w1, dec_b1, dec_w2, dec_b2):
    raise NotImplementedError("write your pallas kernel here")


# speedup vs baseline: 1.6079x; 1.6079x over previous
import jax
import jax.numpy as jnp
from jax.experimental import pallas as pl
from jax.experimental.pallas import tpu as pltpu
from jax.experimental.pallas import tpu_sc as plsc

INPUT_DIM = 1568
NUM_LATENTS = 98
EMBED_DIM = 32
NUM_EMB = 8192
CC = 0.25
BATCH = 1024
HID = 256
N_TOK = BATCH * NUM_LATENTS

T_TOK = 256
C_CHUNK = 1024
N_CHUNKS = NUM_EMB // C_CHUNK
GATHER_W = 128


def _enc_body(x_ref, w1_ref, b1_ref, w2_ref, b2_ref, z_ref):
    h = jnp.dot(x_ref[...], w1_ref[...], preferred_element_type=jnp.float32)
    h = jnp.maximum(h + b1_ref[...], 0.0)
    z_ref[...] = jnp.dot(h, w2_ref[...], preferred_element_type=jnp.float32) + b2_ref[...]


def _vq_body(z_ref, cbt_ref, idx_ref):
    zt = z_ref[...]
    z2 = jnp.sum(zt * zt, axis=1, keepdims=True)
    zs = zt * (-2.0)
    cbt = cbt_ref[...]
    c2 = jnp.sum(cbt * cbt, axis=0, keepdims=True)

    m = jnp.full((T_TOK, C_CHUNK), jnp.inf, jnp.float32)
    cid = jnp.zeros((T_TOK, C_CHUNK), jnp.int32)
    for c in reversed(range(N_CHUNKS)):
        lo = c * C_CHUNK
        mm = jax.lax.dot_general(
            zs, cbt[:, lo:lo + C_CHUNK],
            (((1,), (0,)), ((), ())),
            preferred_element_type=jnp.float32)
        t = z2 + c2[:, lo:lo + C_CHUNK]
        d = t + mm
        m_new = jnp.minimum(m, d)
        cid = jnp.where(m_new == d, jnp.int32(c), cid)
        m = m_new
    rowmin = jnp.min(m, axis=1, keepdims=True)
    lanes = jax.lax.broadcasted_iota(jnp.int32, (T_TOK, C_CHUNK), 1)
    key = jnp.where(m == rowmin, cid * C_CHUNK + lanes, jnp.int32(2 ** 30))
    idx_ref[...] = jnp.min(key, axis=1, keepdims=True)


def _dec_body(z_ref, q_ref, w1_ref, b1_ref, w2_ref, b2_ref,
              xr_ref, vq_ref, pp_ref):
    q = q_ref[...]
    diff = q - z_ref[...]
    mse = jnp.sum(diff * diff) * (1.0 / (BATCH * NUM_LATENTS * EMBED_DIM))
    vq_ref[...] = jnp.reshape((1.0 + CC) * mse, (1, 1))
    p = jnp.float32(1.0 / NUM_EMB)
    pp_ref[...] = jnp.reshape(jnp.exp(-p * jnp.log(p + 1e-10)), (1, 1))
    h = jnp.dot(q, w1_ref[...], preferred_element_type=jnp.float32)
    h = jnp.maximum(h + b1_ref[...], 0.0)
    xr_ref[...] = jnp.dot(h, w2_ref[...], preferred_element_type=jnp.float32) + b2_ref[...]


def _sc_gather(codebook_pad, idx_row):
    vector_mesh = plsc.VectorSubcoreMesh(
        core_axis_name="core", subcore_axis_name="subcore")

    @pl.kernel(
        out_type=jax.ShapeDtypeStruct((N_TOK, 128), jnp.float32),
        mesh=vector_mesh)
    def kern(cb_hbm, i_hbm, o_hbm):
        def body(i_vmem, o_vmem):
            pltpu.sync_copy(cb_hbm.at[i_vmem.at[0]], o_vmem)

        pltpu.emit_pipeline(
            body,
            grid=(N_TOK // GATHER_W,),
            in_specs=[pl.BlockSpec((1, GATHER_W), index_map=lambda i: (0, i))],
            out_specs=[pl.BlockSpec((GATHER_W, 128),
                                    index_map=lambda i: (i, 0))],
            core_axis_name="subcore",
            dimension_semantics=(pltpu.PARALLEL,),
        )(i_hbm, o_hbm)

    return kern(codebook_pad, idx_row)


def kernel(x, enc_w1, enc_b1, enc_w2, enc_b2, codebook,
           dec_w1, dec_b1, dec_w2, dec_b2):
    z = pl.pallas_call(
        _enc_body,
        out_shape=jax.ShapeDtypeStruct((BATCH, NUM_LATENTS * EMBED_DIM), jnp.float32),
    )(x, enc_w1, enc_b1.reshape(1, HID), enc_w2, enc_b2.reshape(1, NUM_LATENTS * EMBED_DIM))

    flat = z.reshape(N_TOK, EMBED_DIM)
    cbt = codebook.T

    idx2d = pl.pallas_call(
        _vq_body,
        grid=(N_TOK // T_TOK,),
        in_specs=[
            pl.BlockSpec((T_TOK, EMBED_DIM), lambda i: (i, 0)),
            pl.BlockSpec((EMBED_DIM, NUM_EMB), lambda i: (0, 0)),
        ],
        out_specs=pl.BlockSpec((T_TOK, 1), lambda i: (i, 0)),
        out_shape=jax.ShapeDtypeStruct((N_TOK, 1), jnp.int32),
    )(flat, cbt)

    cb_pad = jnp.pad(codebook, ((0, 0), (0, 128 - EMBED_DIM)))
    quant = _sc_gather(cb_pad, idx2d.reshape(1, N_TOK))
    q = quant[:, :EMBED_DIM].reshape(BATCH, NUM_LATENTS * EMBED_DIM)

    x_recon, vq_loss, perplexity = pl.pallas_call(
        _dec_body,
        out_shape=(
            jax.ShapeDtypeStruct((BATCH, INPUT_DIM), jnp.float32),
            jax.ShapeDtypeStruct((1, 1), jnp.float32),
            jax.ShapeDtypeStruct((1, 1), jnp.float32),
        ),
    )(z, q, dec_w1, dec_b1.reshape(1, HID), dec_w2, dec_b2.reshape(1, INPUT_DIM))

    encoding_indices = idx2d.reshape(BATCH, NUM_LATENTS, 1)
    return (x_recon, vq_loss[0, 0], perplexity[0, 0], encoding_indices)


# speedup vs baseline: 2.2459x; 1.3968x over previous
import jax
import jax.numpy as jnp
from jax.experimental import pallas as pl
from jax.experimental.pallas import tpu as pltpu
from jax.experimental.pallas import tpu_sc as plsc

INPUT_DIM = 1568
NUM_LATENTS = 98
EMBED_DIM = 32
NUM_EMB = 8192
CC = 0.25
BATCH = 1024
HID = 256
N_TOK = BATCH * NUM_LATENTS

T_TOK = 2048
SB_TOK = 64
C_CHUNK = 256
N_CHUNKS = NUM_EMB // C_CHUNK
GATHER_W = 128


def _enc_body(x_ref, w1_ref, b1_ref, w2_ref, b2_ref, z_ref):
    h = jnp.dot(x_ref[...], w1_ref[...], preferred_element_type=jnp.float32)
    h = jnp.maximum(h + b1_ref[...], 0.0)
    z_ref[...] = jnp.dot(h, w2_ref[...], preferred_element_type=jnp.float32) + b2_ref[...]


def _vq_body(z_ref, cbt_ref, idx_ref):
    cbt = cbt_ref[...]
    c2 = jnp.sum(cbt * cbt, axis=0, keepdims=True)
    cbtb = cbt.astype(jnp.bfloat16)

    for s in range(T_TOK // SB_TOK):
        zt = z_ref[s * SB_TOK:(s + 1) * SB_TOK, :]
        z2 = jnp.sum(zt * zt, axis=1, keepdims=True)
        zs = zt * (-2.0)
        zsb = zs.astype(jnp.bfloat16)

        m = jnp.full((SB_TOK, C_CHUNK), jnp.inf, jnp.float32)
        cid = jnp.zeros((SB_TOK, C_CHUNK), jnp.int32)
        for c in reversed(range(N_CHUNKS)):
            lo = c * C_CHUNK
            mm = jax.lax.dot_general(
                zsb, cbtb[:, lo:lo + C_CHUNK],
                (((1,), (0,)), ((), ())),
                preferred_element_type=jnp.float32)
            t = z2 + c2[:, lo:lo + C_CHUNK]
            d = t + mm
            m_new = jnp.minimum(m, d)
            cid = jnp.where(m_new == d, jnp.int32(c), cid)
            m = m_new
        rowmin = jnp.min(m, axis=1, keepdims=True)
        lanes = jax.lax.broadcasted_iota(jnp.int32, (SB_TOK, C_CHUNK), 1)
        key = jnp.where(m == rowmin, cid * C_CHUNK + lanes, jnp.int32(2 ** 30))
        idx_ref[s * SB_TOK:(s + 1) * SB_TOK, :] = jnp.min(key, axis=1, keepdims=True)


def _dec_body(z_ref, q_ref, w1_ref, b1_ref, w2_ref, b2_ref,
              xr_ref, vq_ref, pp_ref):
    q = q_ref[...]
    diff = q - z_ref[...]
    mse = jnp.sum(diff * diff) * (1.0 / (BATCH * NUM_LATENTS * EMBED_DIM))
    vq_ref[...] = jnp.reshape((1.0 + CC) * mse, (1, 1))
    p = jnp.float32(1.0 / NUM_EMB)
    pp_ref[...] = jnp.reshape(jnp.exp(-p * jnp.log(p + 1e-10)), (1, 1))
    h = jnp.dot(q, w1_ref[...], preferred_element_type=jnp.float32)
    h = jnp.maximum(h + b1_ref[...], 0.0)
    xr_ref[...] = jnp.dot(h, w2_ref[...], preferred_element_type=jnp.float32) + b2_ref[...]


def _sc_gather(codebook_pad, idx_row, n):
    vector_mesh = plsc.VectorSubcoreMesh(
        core_axis_name="core", subcore_axis_name="subcore")

    @pl.kernel(
        out_type=jax.ShapeDtypeStruct((n, 128), jnp.float32),
        mesh=vector_mesh)
    def kern(cb_hbm, i_hbm, o_hbm):
        def body(i_vmem, o_vmem):
            pltpu.sync_copy(cb_hbm.at[i_vmem.at[0]], o_vmem)

        pltpu.emit_pipeline(
            body,
            grid=(n // GATHER_W,),
            in_specs=[pl.BlockSpec((1, GATHER_W), index_map=lambda i: (0, i))],
            out_specs=[pl.BlockSpec((GATHER_W, 128),
                                    index_map=lambda i: (i, 0))],
            core_axis_name="subcore",
            dimension_semantics=(pltpu.PARALLEL,),
        )(i_hbm, o_hbm)

    return kern(codebook_pad, idx_row)


def kernel(x, enc_w1, enc_b1, enc_w2, enc_b2, codebook,
           dec_w1, dec_b1, dec_w2, dec_b2):
    z = pl.pallas_call(
        _enc_body,
        out_shape=jax.ShapeDtypeStruct((BATCH, NUM_LATENTS * EMBED_DIM), jnp.float32),
    )(x, enc_w1, enc_b1.reshape(1, HID), enc_w2, enc_b2.reshape(1, NUM_LATENTS * EMBED_DIM))

    flat = z.reshape(N_TOK, EMBED_DIM)
    cbt = codebook.T
    cb_pad = jnp.pad(codebook, ((0, 0), (0, 128 - EMBED_DIM)))

    idx2d = pl.pallas_call(
        _vq_body,
        grid=(N_TOK // T_TOK,),
        in_specs=[
            pl.BlockSpec((T_TOK, EMBED_DIM), lambda i: (i, 0)),
            pl.BlockSpec((EMBED_DIM, NUM_EMB), lambda i: (0, 0)),
        ],
        out_specs=pl.BlockSpec((T_TOK, 1), lambda i: (i, 0)),
        out_shape=jax.ShapeDtypeStruct((N_TOK, 1), jnp.int32),
    )(flat, cbt)

    quant = _sc_gather(cb_pad, idx2d.reshape(1, N_TOK), N_TOK)
    q = quant[:, :EMBED_DIM].reshape(BATCH, NUM_LATENTS * EMBED_DIM)

    x_recon, vq_loss, perplexity = pl.pallas_call(
        _dec_body,
        out_shape=(
            jax.ShapeDtypeStruct((BATCH, INPUT_DIM), jnp.float32),
            jax.ShapeDtypeStruct((1, 1), jnp.float32),
            jax.ShapeDtypeStruct((1, 1), jnp.float32),
        ),
    )(z, q, dec_w1, dec_b1.reshape(1, HID), dec_w2, dec_b2.reshape(1, INPUT_DIM))

    encoding_indices = idx2d.reshape(BATCH, NUM_LATENTS, 1)
    return (x_recon, vq_loss[0, 0], perplexity[0, 0], encoding_indices)
